# initial kernel scaffold (unmeasured)
import functools

import jax
import jax.numpy as jnp
from jax import lax
from jax.experimental import pallas as pl
from jax.experimental.pallas import tpu as pltpu

NZ = 4


def kernel(x, W, labels):
    T, D = x.shape
    _, Vl = W.shape

    def body(x_ref, w_ref, lab_ref, out_ref, own_ref, comm_ref, send_sems, recv_sems):
        mx = lax.axis_index("x")
        my = lax.axis_index("y")
        mz = lax.axis_index("z")

        barrier = pltpu.get_barrier_semaphore()
        for dz in range(1, NZ):
            pl.semaphore_signal(
                barrier,
                inc=1,
                device_id=(mx, my, (mz + dz) % NZ),
                device_id_type=pl.DeviceIdType.MESH,
            )

        logits = jnp.dot(
            x_ref[:, :], w_ref[:, :], preferred_element_type=jnp.float32
        )
        s = jnp.sum(jnp.exp(logits), axis=1)
        col = lab_ref[:] - mz * Vl
        hit = lax.broadcasted_iota(jnp.int32, (T, Vl), 1) == col[:, None]
        ll = jnp.sum(jnp.where(hit, logits, 0.0), axis=1)
        own_ref[0, :] = s
        own_ref[1, :] = ll

        pl.semaphore_wait(barrier, NZ - 1)

        rdmas = []
        for dz in range(1, NZ):
            rdma = pltpu.make_async_remote_copy(
                src_ref=own_ref,
                dst_ref=comm_ref.at[dz - 1],
                send_sem=send_sems.at[dz - 1],
                recv_sem=recv_sems.at[dz - 1],
                device_id=(mx, my, (mz + dz) % NZ),
                device_id_type=pl.DeviceIdType.MESH,
            )
            rdma.start()
            rdmas.append(rdma)

        for rdma in rdmas:
            rdma.wait_recv()

        s_tot = own_ref[0, :] + comm_ref[0, 0, :] + comm_ref[1, 0, :] + comm_ref[2, 0, :]
        l_tot = own_ref[1, :] + comm_ref[0, 1, :] + comm_ref[1, 1, :] + comm_ref[2, 1, :]
        out_ref[:] = jnp.log(s_tot) - l_tot

        for rdma in rdmas:
            rdma.wait_send()

        @functools.partial(pl.run_scoped, sem2=pltpu.SemaphoreType.REGULAR)
        def _(sem2):
            for dz in range(1, NZ):
                pl.semaphore_signal(
                    sem2,
                    inc=1,
                    device_id=(mx, my, (mz + dz) % NZ),
                    device_id_type=pl.DeviceIdType.MESH,
                )
            pl.semaphore_wait(sem2, NZ - 1)

    return pl.pallas_call(
        body,
        out_shape=jax.ShapeDtypeStruct((T,), jnp.float32),
        in_specs=[
            pl.BlockSpec(memory_space=pltpu.VMEM),
            pl.BlockSpec(memory_space=pltpu.VMEM),
            pl.BlockSpec(memory_space=pltpu.VMEM),
        ],
        out_specs=pl.BlockSpec(memory_space=pltpu.VMEM),
        scratch_shapes=[
            pltpu.VMEM((2, T), jnp.float32),
            pltpu.VMEM((3, 2, T), jnp.float32),
            pltpu.SemaphoreType.DMA((3,)),
            pltpu.SemaphoreType.DMA((3,)),
        ],
        compiler_params=pltpu.CompilerParams(collective_id=0),
    )(x, W, labels)


# baseline (device time: 30443 ns/iter reference)
import functools

import jax
import jax.numpy as jnp
from jax import lax
from jax.experimental import pallas as pl
from jax.experimental.pallas import tpu as pltpu

NZ = 4


def kernel(x, W, labels):
    T, D = x.shape
    _, Vl = W.shape

    def body(x_ref, w_ref, lab_ref, out_ref, own_ref, comm_ref, send_sems, recv_sems):
        mx = lax.axis_index("x")
        my = lax.axis_index("y")
        mz = lax.axis_index("z")

        barrier = pltpu.get_barrier_semaphore()
        for dz in range(1, NZ):
            pl.semaphore_signal(
                barrier,
                inc=1,
                device_id=(mx, my, (mz + dz) % NZ),
                device_id_type=pl.DeviceIdType.MESH,
            )

        logits = jnp.dot(
            x_ref[:, :], w_ref[:, :], preferred_element_type=jnp.float32
        )
        s = jnp.sum(jnp.exp(logits), axis=1)
        col = lab_ref[:] - mz * Vl
        hit = lax.broadcasted_iota(jnp.int32, (T, Vl), 1) == col[:, None]
        ll = jnp.sum(jnp.where(hit, logits, 0.0), axis=1)
        own_ref[0, :] = s
        own_ref[1, :] = ll

        pl.semaphore_wait(barrier, NZ - 1)

        rdmas = []
        for dz in range(1, NZ):
            rdma = pltpu.make_async_remote_copy(
                src_ref=own_ref,
                dst_ref=comm_ref.at[dz - 1],
                send_sem=send_sems.at[dz - 1],
                recv_sem=recv_sems.at[dz - 1],
                device_id=(mx, my, (mz + dz) % NZ),
                device_id_type=pl.DeviceIdType.MESH,
            )
            rdma.start()
            rdmas.append(rdma)

        for rdma in rdmas:
            rdma.wait_recv()

        s_tot = own_ref[0, :] + comm_ref[0, 0, :] + comm_ref[1, 0, :] + comm_ref[2, 0, :]
        l_tot = own_ref[1, :] + comm_ref[0, 1, :] + comm_ref[1, 1, :] + comm_ref[2, 1, :]
        out_ref[:] = jnp.log(s_tot) - l_tot

        for rdma in rdmas:
            rdma.wait_send()

        @functools.partial(pl.run_scoped, sem2=pltpu.SemaphoreType.REGULAR)
        def _(sem2):
            for dz in range(1, NZ):
                pl.semaphore_signal(
                    sem2,
                    inc=1,
                    device_id=(mx, my, (mz + dz) % NZ),
                    device_id_type=pl.DeviceIdType.MESH,
                )
            pl.semaphore_wait(sem2, NZ - 1)

    return pl.pallas_call(
        body,
        out_shape=jax.ShapeDtypeStruct((T,), jnp.float32),
        in_specs=[
            pl.BlockSpec(memory_space=pltpu.VMEM),
            pl.BlockSpec(memory_space=pltpu.VMEM),
            pl.BlockSpec(memory_space=pltpu.VMEM),
        ],
        out_specs=pl.BlockSpec(memory_space=pltpu.VMEM),
        scratch_shapes=[
            pltpu.VMEM((2, T), jnp.float32),
            pltpu.VMEM((3, 2, T), jnp.float32),
            pltpu.SemaphoreType.DMA((3,)),
            pltpu.SemaphoreType.DMA((3,)),
        ],
        compiler_params=pltpu.CompilerParams(
            collective_id=0, vmem_limit_bytes=100 * 1024 * 1024
        ),
    )(x, W, labels)
